# baseline (speedup 1.0000x reference)
import numpy as np

import jax
import jax.numpy as jnp
from jax import lax
from jax.experimental import pallas as pl
from jax.experimental.pallas import tpu as pltpu
from jax.experimental.pallas import tpu_sc as plsc

VOCAB = 1000000
DIM = 16
BATCH = 16384
CTX = 20

NC = 2
NS = 16
NW = NC * NS
B_PER_W = BATCH // NW
CB = 32
NCHUNK = B_PER_W // CB
PAIRS = CB * CTX
GSLICE = 128

NTILE = VOCAB // 128
UNIT = 4
NUNIT = NTILE // UNIT
PROWS = VOCAB // 8


def _pack_body(wtin_hbm, wtout_hbm, tin_hbm, tout_hbm, pin_hbm, pout_hbm,
               buf0, buf1, buf2, buf3, out0, out1, out2, out3,
               sem_in, sem_out):
    wid = lax.axis_index("s") * NC + lax.axis_index("c")
    iota16 = lax.iota(jnp.int32, 16)
    hi = iota16 >> 3
    lo = iota16 & 7
    cols = [lo * 16 + ((lo + (d + 0) + 8 * hi) & 15) for d in range(DIM)]
    nmine = jnp.where(wid < NUNIT - 32 * (NUNIT // 32), NUNIT // 32 + 1,
                      NUNIT // 32)

    def run_table(src_hbm, dst_hbm):
        bufs = (buf0, buf1, buf2, buf3)
        outs = (out0, out1, out2, out3)

        def fetch(c, slot):
            u = wid + 32 * c
            pltpu.async_copy(src_hbm.at[:, pl.ds(u * 512, 512)],
                             bufs[slot], sem_in)

        def drain_in(slot):
            pltpu.make_async_copy(src_hbm.at[:, pl.ds(0, 512)],
                                  bufs[slot], sem_in).wait()

        def flush(c, slot):
            u = wid + 32 * c
            pltpu.async_copy(outs[slot], dst_hbm.at[pl.ds(u * 64, 64), :],
                             sem_out)

        def drain_out(slot):
            pltpu.make_async_copy(outs[slot],
                                  dst_hbm.at[pl.ds(0, 64), :], sem_out).wait()

        def permute(slot):
            buf, out = bufs[slot], outs[slot]

            def k_body(kp, carry):
                for kk in range(4):
                    k = 4 * kp + kk
                    rowv = hi + 2 * k
                    for d in range(DIM):
                        v = buf[d, pl.ds(16 * k, 16)]
                        plsc.store_scatter(out, [rowv, cols[d]], v)
                return carry

            lax.fori_loop(0, 8, k_body, 0)

        for i in range(3):
            @pl.when(i < nmine)
            def _(i=i):
                fetch(i, i)

        def quad_body(p, carry):
            c0 = 4 * p
            for i in range(4):
                @pl.when(c0 + i < nmine)
                def _(i=i):
                    @pl.when(c0 + i + 3 < nmine)
                    def _():
                        fetch(c0 + i + 3, (i + 3) % 4)
                    drain_in(i)

                    @pl.when(c0 + i >= 4)
                    def _():
                        drain_out(i)
                    permute(i)
                    flush(c0 + i, i)

            return carry

        lax.fori_loop(0, (NUNIT // 32 + 4) // 4, quad_body, 0)

        for i in range(4):
            @pl.when(i < nmine)
            def _(i=i):
                drain_out(i)

    run_table(wtin_hbm, pin_hbm)
    run_table(wtout_hbm, pout_hbm)

    @pl.when(wid == 0)
    def _():
        pltpu.sync_copy(tin_hbm, buf0.at[pl.ds(0, 8), pl.ds(0, 128)])
        pltpu.sync_copy(buf0.at[pl.ds(0, 8), pl.ds(0, 128)],
                        pin_hbm.at[pl.ds(NTILE * 16, 8), :])
        pltpu.sync_copy(tout_hbm, buf1.at[pl.ds(0, 8), pl.ds(0, 128)])
        pltpu.sync_copy(buf1.at[pl.ds(0, 8), pl.ds(0, 128)],
                        pout_hbm.at[pl.ds(NTILE * 16, 8), :])


def _dot_body(frow_hbm, fsub_hbm, frot_hbm, crow_hbm, csub_hbm, crot_hbm,
              pin_hbm, pout_hbm, out_hbm,
              idx_f, sub_f, rot_f, idx_c, sub_c, rot_c,
              frows, crows, out_v, sem):
    wid = lax.axis_index("s") * NC + lax.axis_index("c")
    iota16 = lax.iota(jnp.int32, 16)

    def chunk_body(c, carry):
        chunk = wid * NCHUNK + c
        pltpu.sync_copy(frow_hbm.at[pl.ds(chunk * CB, CB)], idx_f)
        pltpu.sync_copy(fsub_hbm.at[pl.ds(chunk * CB, CB)], sub_f)
        pltpu.sync_copy(frot_hbm.at[pl.ds(chunk * CB, CB)], rot_f)
        pltpu.sync_copy(crow_hbm.at[pl.ds(chunk * PAIRS, PAIRS)], idx_c)
        pltpu.sync_copy(csub_hbm.at[pl.ds(chunk * PAIRS, PAIRS)], sub_c)
        pltpu.sync_copy(crot_hbm.at[pl.ds(chunk * PAIRS, PAIRS)], rot_c)
        copies = [pltpu.async_copy(pin_hbm.at[idx_f], frows, sem)]
        for j in range(PAIRS // GSLICE):
            copies.append(pltpu.async_copy(
                pout_hbm.at[idx_c.at[pl.ds(j * GSLICE, GSLICE)]],
                crows.at[pl.ds(j * GSLICE, GSLICE)], sem))
        for cp in copies:
            cp.wait()

        def g_body(g, carry2):
            bvec = g * 16 + iota16
            fsubv = plsc.load_gather(sub_f, [bvec])
            frotv = plsc.load_gather(rot_f, [bvec])
            fcols = [plsc.load_gather(
                frows, [bvec, fsubv + ((frotv + d) & 15)])
                for d in range(DIM)]
            base = bvec * CTX

            def l_body(l, carry3):
                pvec = base + l
                csubv = plsc.load_gather(sub_c, [pvec])
                crotv = plsc.load_gather(rot_c, [pvec])
                acc = jnp.zeros((16,), jnp.float32)
                for d in range(DIM):
                    cv = plsc.load_gather(
                        crows, [pvec, csubv + ((crotv + d) & 15)])
                    acc = acc + cv * fcols[d]
                plsc.store_scatter(out_v, [pvec], acc)
                return carry3

            lax.fori_loop(0, CTX, l_body, 0)
            return carry2

        lax.fori_loop(0, CB // 16, g_body, 0)
        pltpu.sync_copy(out_v, out_hbm.at[pl.ds(chunk * PAIRS, PAIRS)])
        return carry

    lax.fori_loop(0, NCHUNK, chunk_body, 0)


def _pack_tail(w):
    j = np.arange(8)[:, None]
    c = np.arange(128)[None, :]
    s = c // 16
    d = (c % 16 - s - 8 * (j & 1)) % 16
    r = 8 * j + s
    return w[jnp.asarray(r), jnp.asarray(d)]


def kernel(focus_item_batch, context_items_batch, W_in, W_out):
    focus = focus_item_batch.reshape(BATCH).astype(jnp.int32)
    ctx = context_items_batch.reshape(BATCH * CTX).astype(jnp.int32)
    frow = focus >> 3
    fsub = (focus & 7) * DIM
    frot = (focus & 7) + ((focus >> 3) & 1) * 8
    crow = ctx >> 3
    csub = (ctx & 7) * DIM
    crot = (ctx & 7) + ((ctx >> 3) & 1) * 8

    tin = _pack_tail(W_in[NTILE * 128:])
    tout = _pack_tail(W_out[NTILE * 128:])

    pack = pl.kernel(
        _pack_body,
        out_type=(
            jax.ShapeDtypeStruct((PROWS, 128), jnp.float32),
            jax.ShapeDtypeStruct((PROWS, 128), jnp.float32),
        ),
        mesh=plsc.VectorSubcoreMesh(core_axis_name="c", subcore_axis_name="s"),
        compiler_params=pltpu.CompilerParams(
            needs_layout_passes=False, use_tc_tiling_on_sc=True),
        scratch_types=[
            pltpu.VMEM((16, 512), jnp.float32),
            pltpu.VMEM((16, 512), jnp.float32),
            pltpu.VMEM((16, 512), jnp.float32),
            pltpu.VMEM((16, 512), jnp.float32),
            pltpu.VMEM((64, 128), jnp.float32),
            pltpu.VMEM((64, 128), jnp.float32),
            pltpu.VMEM((64, 128), jnp.float32),
            pltpu.VMEM((64, 128), jnp.float32),
            pltpu.SemaphoreType.DMA,
            pltpu.SemaphoreType.DMA,
        ],
    )
    pin, pout = pack(W_in.T, W_out.T, tin, tout)

    run = pl.kernel(
        _dot_body,
        out_type=jax.ShapeDtypeStruct((BATCH * CTX,), jnp.float32),
        mesh=plsc.VectorSubcoreMesh(core_axis_name="c", subcore_axis_name="s"),
        compiler_params=pltpu.CompilerParams(
            needs_layout_passes=False, use_tc_tiling_on_sc=True),
        scratch_types=[
            pltpu.VMEM((CB,), jnp.int32),
            pltpu.VMEM((CB,), jnp.int32),
            pltpu.VMEM((CB,), jnp.int32),
            pltpu.VMEM((PAIRS,), jnp.int32),
            pltpu.VMEM((PAIRS,), jnp.int32),
            pltpu.VMEM((PAIRS,), jnp.int32),
            pltpu.VMEM((CB, 128), jnp.float32),
            pltpu.VMEM((PAIRS, 128), jnp.float32),
            pltpu.VMEM((PAIRS,), jnp.float32),
            pltpu.SemaphoreType.DMA,
        ],
    )
    out = run(frow, fsub, frot, crow, csub, crot, pin, pout)
    return out.reshape(BATCH, CTX)


# speedup vs baseline: 1.0233x; 1.0233x over previous
import numpy as np

import jax
import jax.numpy as jnp
from jax import lax
from jax.experimental import pallas as pl
from jax.experimental.pallas import tpu as pltpu
from jax.experimental.pallas import tpu_sc as plsc

VOCAB = 1000000
DIM = 16
BATCH = 16384
CTX = 20

NC = 2
NS = 16
NW = NC * NS
B_PER_W = BATCH // NW
CB = 32
NCHUNK = B_PER_W // CB
PAIRS = CB * CTX
GSLICE = 128

NTILE = VOCAB // 128
UNIT = 4
NUNIT = NTILE // UNIT
PROWS = VOCAB // 8


def _pack_body(wtin_hbm, wtout_hbm, tin_hbm, tout_hbm, pin_hbm, pout_hbm,
               buf0, buf1, buf2, buf3, out0, out1, out2, out3,
               sem_in, sem_out):
    wid = lax.axis_index("s") * NC + lax.axis_index("c")
    iota16 = lax.iota(jnp.int32, 16)
    hi = iota16 >> 3
    lo = iota16 & 7
    cols = [lo * 16 + ((lo + (d + 0) + 8 * hi) & 15) for d in range(DIM)]
    nmine = jnp.where(wid < NUNIT - 32 * (NUNIT // 32), NUNIT // 32 + 1,
                      NUNIT // 32)

    def run_table(src_hbm, dst_hbm):
        bufs = (buf0, buf1, buf2, buf3)
        outs = (out0, out1, out2, out3)

        def fetch(c, slot):
            u = wid + 32 * c
            pltpu.async_copy(src_hbm.at[:, pl.ds(u * 512, 512)],
                             bufs[slot], sem_in)

        def drain_in(slot):
            pltpu.make_async_copy(src_hbm.at[:, pl.ds(0, 512)],
                                  bufs[slot], sem_in).wait()

        def flush(c, slot):
            u = wid + 32 * c
            pltpu.async_copy(outs[slot], dst_hbm.at[pl.ds(u * 64, 64), :],
                             sem_out)

        def drain_out(slot):
            pltpu.make_async_copy(outs[slot],
                                  dst_hbm.at[pl.ds(0, 64), :], sem_out).wait()

        def permute(slot):
            buf, out = bufs[slot], outs[slot]

            def k_body(kp, carry):
                for kk in range(2):
                    k = 2 * kp + kk
                    rowv = hi + 2 * k
                    for d in range(DIM):
                        v = buf[d, pl.ds(16 * k, 16)]
                        plsc.store_scatter(out, [rowv, cols[d]], v)
                return carry

            lax.fori_loop(0, 16, k_body, 0)

        for i in range(3):
            @pl.when(i < nmine)
            def _(i=i):
                fetch(i, i)

        def quad_body(p, carry):
            c0 = 4 * p
            for i in range(4):
                @pl.when(c0 + i < nmine)
                def _(i=i):
                    @pl.when(c0 + i + 3 < nmine)
                    def _():
                        fetch(c0 + i + 3, (i + 3) % 4)
                    drain_in(i)

                    @pl.when(c0 + i >= 4)
                    def _():
                        drain_out(i)
                    permute(i)
                    flush(c0 + i, i)

            return carry

        lax.fori_loop(0, (NUNIT // 32 + 4) // 4, quad_body, 0)

        for i in range(4):
            @pl.when(i < nmine)
            def _(i=i):
                drain_out(i)

    run_table(wtin_hbm, pin_hbm)
    run_table(wtout_hbm, pout_hbm)

    @pl.when(wid == 0)
    def _():
        pltpu.sync_copy(tin_hbm, buf0.at[pl.ds(0, 8), pl.ds(0, 128)])
        pltpu.sync_copy(buf0.at[pl.ds(0, 8), pl.ds(0, 128)],
                        pin_hbm.at[pl.ds(NTILE * 16, 8), :])
        pltpu.sync_copy(tout_hbm, buf1.at[pl.ds(0, 8), pl.ds(0, 128)])
        pltpu.sync_copy(buf1.at[pl.ds(0, 8), pl.ds(0, 128)],
                        pout_hbm.at[pl.ds(NTILE * 16, 8), :])


def _dot_body(frow_hbm, fsub_hbm, frot_hbm, crow_hbm, csub_hbm, crot_hbm,
              pin_hbm, pout_hbm, out_hbm,
              idx_f, sub_f, rot_f, idx_c, sub_c, rot_c,
              frows, crows, out_v, sem):
    wid = lax.axis_index("s") * NC + lax.axis_index("c")
    iota16 = lax.iota(jnp.int32, 16)

    def chunk_body(c, carry):
        chunk = wid * NCHUNK + c
        pltpu.sync_copy(frow_hbm.at[pl.ds(chunk * CB, CB)], idx_f)
        pltpu.sync_copy(fsub_hbm.at[pl.ds(chunk * CB, CB)], sub_f)
        pltpu.sync_copy(frot_hbm.at[pl.ds(chunk * CB, CB)], rot_f)
        pltpu.sync_copy(crow_hbm.at[pl.ds(chunk * PAIRS, PAIRS)], idx_c)
        pltpu.sync_copy(csub_hbm.at[pl.ds(chunk * PAIRS, PAIRS)], sub_c)
        pltpu.sync_copy(crot_hbm.at[pl.ds(chunk * PAIRS, PAIRS)], rot_c)
        copies = [pltpu.async_copy(pin_hbm.at[idx_f], frows, sem)]
        for j in range(PAIRS // GSLICE):
            copies.append(pltpu.async_copy(
                pout_hbm.at[idx_c.at[pl.ds(j * GSLICE, GSLICE)]],
                crows.at[pl.ds(j * GSLICE, GSLICE)], sem))
        for cp in copies:
            cp.wait()

        def g_body(g, carry2):
            bvec = g * 16 + iota16
            fsubv = plsc.load_gather(sub_f, [bvec])
            frotv = plsc.load_gather(rot_f, [bvec])
            fcols = [plsc.load_gather(
                frows, [bvec, fsubv + ((frotv + d) & 15)])
                for d in range(DIM)]
            base = bvec * CTX

            def l_body(l, carry3):
                pvec = base + l
                csubv = plsc.load_gather(sub_c, [pvec])
                crotv = plsc.load_gather(rot_c, [pvec])
                acc = jnp.zeros((16,), jnp.float32)
                for d in range(DIM):
                    cv = plsc.load_gather(
                        crows, [pvec, csubv + ((crotv + d) & 15)])
                    acc = acc + cv * fcols[d]
                plsc.store_scatter(out_v, [pvec], acc)
                return carry3

            lax.fori_loop(0, CTX, l_body, 0)
            return carry2

        lax.fori_loop(0, CB // 16, g_body, 0)
        pltpu.sync_copy(out_v, out_hbm.at[pl.ds(chunk * PAIRS, PAIRS)])
        return carry

    lax.fori_loop(0, NCHUNK, chunk_body, 0)


def _pack_tail(w):
    j = np.arange(8)[:, None]
    c = np.arange(128)[None, :]
    s = c // 16
    d = (c % 16 - s - 8 * (j & 1)) % 16
    r = 8 * j + s
    return w[jnp.asarray(r), jnp.asarray(d)]


def kernel(focus_item_batch, context_items_batch, W_in, W_out):
    focus = focus_item_batch.reshape(BATCH).astype(jnp.int32)
    ctx = context_items_batch.reshape(BATCH * CTX).astype(jnp.int32)
    frow = focus >> 3
    fsub = (focus & 7) * DIM
    frot = (focus & 7) + ((focus >> 3) & 1) * 8
    crow = ctx >> 3
    csub = (ctx & 7) * DIM
    crot = (ctx & 7) + ((ctx >> 3) & 1) * 8

    tin = _pack_tail(W_in[NTILE * 128:])
    tout = _pack_tail(W_out[NTILE * 128:])

    pack = pl.kernel(
        _pack_body,
        out_type=(
            jax.ShapeDtypeStruct((PROWS, 128), jnp.float32),
            jax.ShapeDtypeStruct((PROWS, 128), jnp.float32),
        ),
        mesh=plsc.VectorSubcoreMesh(core_axis_name="c", subcore_axis_name="s"),
        compiler_params=pltpu.CompilerParams(
            needs_layout_passes=False, use_tc_tiling_on_sc=True),
        scratch_types=[
            pltpu.VMEM((16, 512), jnp.float32),
            pltpu.VMEM((16, 512), jnp.float32),
            pltpu.VMEM((16, 512), jnp.float32),
            pltpu.VMEM((16, 512), jnp.float32),
            pltpu.VMEM((64, 128), jnp.float32),
            pltpu.VMEM((64, 128), jnp.float32),
            pltpu.VMEM((64, 128), jnp.float32),
            pltpu.VMEM((64, 128), jnp.float32),
            pltpu.SemaphoreType.DMA,
            pltpu.SemaphoreType.DMA,
        ],
    )
    pin, pout = pack(W_in.T, W_out.T, tin, tout)

    run = pl.kernel(
        _dot_body,
        out_type=jax.ShapeDtypeStruct((BATCH * CTX,), jnp.float32),
        mesh=plsc.VectorSubcoreMesh(core_axis_name="c", subcore_axis_name="s"),
        compiler_params=pltpu.CompilerParams(
            needs_layout_passes=False, use_tc_tiling_on_sc=True),
        scratch_types=[
            pltpu.VMEM((CB,), jnp.int32),
            pltpu.VMEM((CB,), jnp.int32),
            pltpu.VMEM((CB,), jnp.int32),
            pltpu.VMEM((PAIRS,), jnp.int32),
            pltpu.VMEM((PAIRS,), jnp.int32),
            pltpu.VMEM((PAIRS,), jnp.int32),
            pltpu.VMEM((CB, 128), jnp.float32),
            pltpu.VMEM((PAIRS, 128), jnp.float32),
            pltpu.VMEM((PAIRS,), jnp.float32),
            pltpu.SemaphoreType.DMA,
        ],
    )
    out = run(frow, fsub, frot, crow, csub, crot, pin, pout)
    return out.reshape(BATCH, CTX)


# speedup vs baseline: 1.1224x; 1.0969x over previous
import numpy as np

import jax
import jax.numpy as jnp
from jax import lax
from jax.experimental import pallas as pl
from jax.experimental.pallas import tpu as pltpu
from jax.experimental.pallas import tpu_sc as plsc

VOCAB = 1000000
DIM = 16
BATCH = 16384
CTX = 20

NC = 2
NS = 16
NW = NC * NS
B_PER_W = BATCH // NW
CB = 32
NCHUNK = B_PER_W // CB
PAIRS = CB * CTX
GSLICE = 128

NTILE = VOCAB // 128
UNIT = 4
NUNIT = NTILE // UNIT
PROWS = VOCAB // 8


def _pack_body(wtin_hbm, wtout_hbm, tin_hbm, tout_hbm, pin_hbm, pout_hbm,
               buf0, buf1, buf2, buf3, out0, out1, out2, out3,
               sem_in, sem_out):
    wid = lax.axis_index("s") * NC + lax.axis_index("c")
    iota16 = lax.iota(jnp.int32, 16)
    hi = iota16 >> 3
    lo = iota16 & 7
    cols = [lo * 16 + ((lo + (d + 0) + 8 * hi) & 15) for d in range(DIM)]
    nmine = jnp.where(wid < NUNIT - 32 * (NUNIT // 32), NUNIT // 32 + 1,
                      NUNIT // 32)

    def run_table(src_hbm, dst_hbm):
        bufs = (buf0, buf1, buf2, buf3)
        outs = (out0, out1, out2, out3)

        def fetch(c, slot):
            u = wid + 32 * c
            pltpu.async_copy(src_hbm.at[:, pl.ds(u * 512, 512)],
                             bufs[slot], sem_in)

        def drain_in(slot):
            pltpu.make_async_copy(src_hbm.at[:, pl.ds(0, 512)],
                                  bufs[slot], sem_in).wait()

        def flush(c, slot):
            u = wid + 32 * c
            pltpu.async_copy(outs[slot], dst_hbm.at[pl.ds(u * 64, 64), :],
                             sem_out)

        def drain_out(slot):
            pltpu.make_async_copy(outs[slot],
                                  dst_hbm.at[pl.ds(0, 64), :], sem_out).wait()

        def permute(slot):
            buf, out = bufs[slot], outs[slot]

            def k_body(kp, carry):
                for kk in range(2):
                    k = 2 * kp + kk
                    rowv = hi + 2 * k
                    for d in range(DIM):
                        v = buf[d, pl.ds(16 * k, 16)]
                        plsc.store_scatter(out, [rowv, cols[d]], v)
                return carry

            lax.fori_loop(0, 16, k_body, 0)

        for i in range(3):
            @pl.when(i < nmine)
            def _(i=i):
                fetch(i, i)

        def quad_body(p, carry):
            c0 = 4 * p
            for i in range(4):
                @pl.when(c0 + i < nmine)
                def _(i=i):
                    @pl.when(c0 + i + 3 < nmine)
                    def _():
                        fetch(c0 + i + 3, (i + 3) % 4)
                    drain_in(i)

                    @pl.when(c0 + i >= 4)
                    def _():
                        drain_out(i)
                    permute(i)
                    flush(c0 + i, i)

            return carry

        lax.fori_loop(0, (NUNIT // 32 + 4) // 4, quad_body, 0)

        for i in range(4):
            @pl.when(i < nmine)
            def _(i=i):
                drain_out(i)

    run_table(wtin_hbm, pin_hbm)
    run_table(wtout_hbm, pout_hbm)

    @pl.when(wid == 0)
    def _():
        pltpu.sync_copy(tin_hbm, buf0.at[pl.ds(0, 8), pl.ds(0, 128)])
        pltpu.sync_copy(buf0.at[pl.ds(0, 8), pl.ds(0, 128)],
                        pin_hbm.at[pl.ds(NTILE * 16, 8), :])
        pltpu.sync_copy(tout_hbm, buf1.at[pl.ds(0, 8), pl.ds(0, 128)])
        pltpu.sync_copy(buf1.at[pl.ds(0, 8), pl.ds(0, 128)],
                        pout_hbm.at[pl.ds(NTILE * 16, 8), :])


def _dot_body(frow_hbm, fsub_hbm, frot_hbm, crow_hbm, csub_hbm, crot_hbm,
              pin_hbm, pout_hbm, out_hbm,
              idx_f, sub_f, rot_f, idx_c, sub_c, rot_c,
              frows, crows, out_v, sem):
    wid = lax.axis_index("s") * NC + lax.axis_index("c")
    iota16 = lax.iota(jnp.int32, 16)

    def chunk_body(c, carry):
        chunk = wid * NCHUNK + c
        stage = [
            pltpu.async_copy(frow_hbm.at[pl.ds(chunk * CB, CB)], idx_f, sem),
            pltpu.async_copy(fsub_hbm.at[pl.ds(chunk * CB, CB)], sub_f, sem),
            pltpu.async_copy(frot_hbm.at[pl.ds(chunk * CB, CB)], rot_f, sem),
            pltpu.async_copy(crow_hbm.at[pl.ds(chunk * PAIRS, PAIRS)], idx_c, sem),
            pltpu.async_copy(csub_hbm.at[pl.ds(chunk * PAIRS, PAIRS)], sub_c, sem),
            pltpu.async_copy(crot_hbm.at[pl.ds(chunk * PAIRS, PAIRS)], rot_c, sem),
        ]
        for cp in stage:
            cp.wait()
        copies = [pltpu.async_copy(pin_hbm.at[idx_f], frows, sem)]
        for j in range(PAIRS // GSLICE):
            copies.append(pltpu.async_copy(
                pout_hbm.at[idx_c.at[pl.ds(j * GSLICE, GSLICE)]],
                crows.at[pl.ds(j * GSLICE, GSLICE)], sem))
        for cp in copies:
            cp.wait()

        def g_body(g, carry2):
            bvec = g * 16 + iota16
            fsubv = plsc.load_gather(sub_f, [bvec])
            frotv = plsc.load_gather(rot_f, [bvec])
            fcols = [plsc.load_gather(
                frows, [bvec, fsubv + ((frotv + d) & 15)])
                for d in range(DIM)]
            base = bvec * CTX

            def l_body(l, carry3):
                pvec = base + l
                csubv = plsc.load_gather(sub_c, [pvec])
                crotv = plsc.load_gather(rot_c, [pvec])
                acc = jnp.zeros((16,), jnp.float32)
                for d in range(DIM):
                    cv = plsc.load_gather(
                        crows, [pvec, csubv + ((crotv + d) & 15)])
                    acc = acc + cv * fcols[d]
                plsc.store_scatter(out_v, [pvec], acc)
                return carry3

            lax.fori_loop(0, CTX, l_body, 0)
            return carry2

        lax.fori_loop(0, CB // 16, g_body, 0)
        pltpu.sync_copy(out_v, out_hbm.at[pl.ds(chunk * PAIRS, PAIRS)])
        return carry

    lax.fori_loop(0, NCHUNK, chunk_body, 0)


def _pack_tail(w):
    j = np.arange(8)[:, None]
    c = np.arange(128)[None, :]
    s = c // 16
    d = (c % 16 - s - 8 * (j & 1)) % 16
    r = 8 * j + s
    return w[jnp.asarray(r), jnp.asarray(d)]


def kernel(focus_item_batch, context_items_batch, W_in, W_out):
    focus = focus_item_batch.reshape(BATCH).astype(jnp.int32)
    ctx = context_items_batch.reshape(BATCH * CTX).astype(jnp.int32)
    frow = focus >> 3
    fsub = (focus & 7) * DIM
    frot = (focus & 7) + ((focus >> 3) & 1) * 8
    crow = ctx >> 3
    csub = (ctx & 7) * DIM
    crot = (ctx & 7) + ((ctx >> 3) & 1) * 8

    tin = _pack_tail(W_in[NTILE * 128:])
    tout = _pack_tail(W_out[NTILE * 128:])

    pack = pl.kernel(
        _pack_body,
        out_type=(
            jax.ShapeDtypeStruct((PROWS, 128), jnp.float32),
            jax.ShapeDtypeStruct((PROWS, 128), jnp.float32),
        ),
        mesh=plsc.VectorSubcoreMesh(core_axis_name="c", subcore_axis_name="s"),
        compiler_params=pltpu.CompilerParams(
            needs_layout_passes=False, use_tc_tiling_on_sc=True),
        scratch_types=[
            pltpu.VMEM((16, 512), jnp.float32),
            pltpu.VMEM((16, 512), jnp.float32),
            pltpu.VMEM((16, 512), jnp.float32),
            pltpu.VMEM((16, 512), jnp.float32),
            pltpu.VMEM((64, 128), jnp.float32),
            pltpu.VMEM((64, 128), jnp.float32),
            pltpu.VMEM((64, 128), jnp.float32),
            pltpu.VMEM((64, 128), jnp.float32),
            pltpu.SemaphoreType.DMA,
            pltpu.SemaphoreType.DMA,
        ],
    )
    pin, pout = pack(W_in.T, W_out.T, tin, tout)

    run = pl.kernel(
        _dot_body,
        out_type=jax.ShapeDtypeStruct((BATCH * CTX,), jnp.float32),
        mesh=plsc.VectorSubcoreMesh(core_axis_name="c", subcore_axis_name="s"),
        compiler_params=pltpu.CompilerParams(
            needs_layout_passes=False, use_tc_tiling_on_sc=True),
        scratch_types=[
            pltpu.VMEM((CB,), jnp.int32),
            pltpu.VMEM((CB,), jnp.int32),
            pltpu.VMEM((CB,), jnp.int32),
            pltpu.VMEM((PAIRS,), jnp.int32),
            pltpu.VMEM((PAIRS,), jnp.int32),
            pltpu.VMEM((PAIRS,), jnp.int32),
            pltpu.VMEM((CB, 128), jnp.float32),
            pltpu.VMEM((PAIRS, 128), jnp.float32),
            pltpu.VMEM((PAIRS,), jnp.float32),
            pltpu.SemaphoreType.DMA,
        ],
    )
    out = run(frow, fsub, frot, crow, csub, crot, pin, pout)
    return out.reshape(BATCH, CTX)
